# Initial kernel scaffold; baseline (speedup 1.0000x reference)
import jax
import jax.numpy as jnp
from jax.experimental import pallas as pl


def kernel(x, table):
    raise NotImplementedError("write your pallas kernel here")


# speedup vs baseline: 1.0905x; 1.0905x over previous
import functools

import jax
import jax.numpy as jnp
from jax import lax
from jax.experimental import pallas as pl
from jax.experimental.pallas import tpu as pltpu
from jax.experimental.pallas import tpu_sc as plsc

_D = 2048
_NC = 2
_NS = 16
_NW = _NC * _NS
_W = 16
_NBUF = 2


def kernel(x, table):
    b0, b1 = x.shape
    num = b0 * b1
    bpw = num // _NW
    nchunk = bpw // _W
    idx = x.reshape(num)

    mesh = plsc.VectorSubcoreMesh(core_axis_name="c", subcore_axis_name="s")

    @functools.partial(
        pl.kernel,
        mesh=mesh,
        out_type=jax.ShapeDtypeStruct((num, _D), table.dtype),
        scratch_types=[
            pltpu.VMEM((bpw,), jnp.int32),
            pltpu.VMEM((_W, _D), jnp.float32),
            pltpu.VMEM((_W, _D), jnp.float32),
            pltpu.SemaphoreType.DMA,
            pltpu.SemaphoreType.DMA,
        ],
    )
    def run(table_hbm, idx_hbm, out_hbm, idx_v, buf0, buf1, sem0, sem1):
        wid = lax.axis_index("s") * _NC + lax.axis_index("c")
        base = wid * bpw
        pltpu.sync_copy(idx_hbm.at[pl.ds(base, bpw)], idx_v)

        bufs = (buf0, buf1)
        sems = (sem0, sem1)
        for b in range(_NBUF):
            pltpu.async_copy(
                table_hbm.at[idx_v.at[pl.ds(b * _W, _W)]], bufs[b], sems[b]
            )

        @pl.loop(0, nchunk, step=_NBUF)
        def _(j):
            for b in range(_NBUF):
                c = j + b
                pltpu.make_async_copy(
                    table_hbm.at[idx_v.at[pl.ds(c * _W, _W)]], bufs[b], sems[b]
                ).wait()
                pltpu.sync_copy(bufs[b], out_hbm.at[pl.ds(base + c * _W, _W)])

                @pl.when(c + _NBUF < nchunk)
                def _():
                    pltpu.async_copy(
                        table_hbm.at[idx_v.at[pl.ds((c + _NBUF) * _W, _W)]],
                        bufs[b],
                        sems[b],
                    )

    out = run(table, idx)
    return out.reshape(b0, b1, _D)


# speedup vs baseline: 2.6283x; 2.4102x over previous
import functools

import jax
import jax.numpy as jnp
from jax import lax
from jax.experimental import pallas as pl
from jax.experimental.pallas import tpu as pltpu
from jax.experimental.pallas import tpu_sc as plsc

_D = 2048
_NC = 2
_NS = 16
_NW = _NC * _NS
_W = 16
_NBUF = 2


def kernel(x, table):
    b0, b1 = x.shape
    num = b0 * b1
    bpw = num // _NW
    nchunk = bpw // _W
    idx = x.T.reshape(num)

    mesh = plsc.VectorSubcoreMesh(core_axis_name="c", subcore_axis_name="s")

    @functools.partial(
        pl.kernel,
        mesh=mesh,
        out_type=jax.ShapeDtypeStruct((num, _D), table.dtype),
        scratch_types=[
            pltpu.VMEM((bpw,), jnp.int32),
            pltpu.VMEM((_W, _D), jnp.float32),
            pltpu.VMEM((_W, _D), jnp.float32),
            pltpu.SemaphoreType.DMA,
            pltpu.SemaphoreType.DMA,
        ],
    )
    def run(table_hbm, idx_hbm, out_hbm, idx_v, buf0, buf1, sem0, sem1):
        wid = lax.axis_index("s") * _NC + lax.axis_index("c")
        base = wid * bpw
        pltpu.sync_copy(idx_hbm.at[pl.ds(base, bpw)], idx_v)

        bufs = (buf0, buf1)
        sems = (sem0, sem1)
        for b in range(_NBUF):
            pltpu.async_copy(
                table_hbm.at[idx_v.at[pl.ds(b * _W, _W)]], bufs[b], sems[b]
            )

        @pl.loop(0, nchunk, step=_NBUF)
        def _(j):
            for b in range(_NBUF):
                c = j + b
                pltpu.make_async_copy(
                    table_hbm.at[idx_v.at[pl.ds(c * _W, _W)]], bufs[b], sems[b]
                ).wait()
                pltpu.sync_copy(bufs[b], out_hbm.at[pl.ds(base + c * _W, _W)])

                @pl.when(c + _NBUF < nchunk)
                def _():
                    pltpu.async_copy(
                        table_hbm.at[idx_v.at[pl.ds((c + _NBUF) * _W, _W)]],
                        bufs[b],
                        sems[b],
                    )

    out = run(table, idx)
    return out.reshape(b1, b0, _D).transpose(1, 0, 2)
